# baseline (speedup 1.0000x reference)
import functools

import jax
import jax.numpy as jnp
from jax import lax
from jax.experimental import pallas as pl
from jax.experimental.pallas import tpu as pltpu

_GELU_C = (0.3989401215833544, -1.0637133263229617, 2.55036139654138,
           -4.832030293421218, 7.364302003605586, -9.075135234692333,
           8.861428611075373, -6.545424961401749, 3.391661863797997,
           -1.0852160629273473, 0.15981801221487357)


def _gelu(x):
    s = x * x * 0.0625
    p = jnp.float32(_GELU_C[-1])
    for c in _GELU_C[-2::-1]:
        p = p * s + c
    core = 0.5 * x + (x * x) * p
    return jnp.where(x > 4.0, x, jnp.where(x < -4.0, 0.0, core))


def _w3x3_to_mat(w):
    co = w.shape[0]
    return jnp.transpose(w, (0, 2, 3, 1)).reshape(co, 9 * w.shape[1])


def _lo16(x):
    return lax.bitcast_convert_type(x.astype(jnp.int16), jnp.bfloat16)


def _hi16(x):
    return lax.bitcast_convert_type(
        lax.shift_right_logical(x, jnp.int32(16)).astype(jnp.int16),
        jnp.bfloat16)


def _pack16(lo, hi):
    lo_u = lax.bitcast_convert_type(lo, jnp.uint16).astype(jnp.uint32)
    hi_u = lax.bitcast_convert_type(hi, jnp.uint16).astype(jnp.uint32)
    return (lo_u | (hi_u << 16)).astype(jnp.int32)


def _fused_kernel(xe_ref, xo_ref, x2_ref, whq_ref, wlq_ref, wx2_ref, bh1_ref,
                  bl1_ref, wha_ref, bha_ref, whb_ref, bhb_ref, wla_ref,
                  bla_ref, wlb_ref, blb_ref, whd_ref, bhd_ref, o_ref,
                  qt_s, x0_s, sx_s, h1_s, hi_s, lo_s, *, C, TH, nT, W2, H2):
    Lx = (TH + 4) * W2
    L1 = (TH + 2) * W2
    L2 = TH * W2
    bf = jnp.bfloat16

    col = lax.broadcasted_iota(jnp.int32, (1, Lx), 1) % W2
    cm_m = (col != 0).astype(bf)
    cm_p = (col != W2 - 1).astype(bf)
    frow = lax.broadcasted_iota(jnp.int32, (1, Lx), 1) // W2

    def row_mask(x, t, start_f):
        if 0 < t < nT - 1:
            return x
        n = x.shape[1] // W2
        orig = t * TH + start_f - 3 + frow[:, :n * W2]
        return jnp.where((orig >= 0) & (orig < H2), x, 0.0)

    def shifts(v, L):
        sm = jnp.concatenate([v[:, :1], v[:, :-1]], axis=-1) * cm_m[:, :L]
        sp = jnp.concatenate([v[:, 1:], v[:, -1:]], axis=-1) * cm_p[:, :L]
        return sm, sp

    def conv3x3(src_len, out_len, w_ref, b_ref, src_sref):
        v = src_sref[:, : src_len]
        sm, sp = shifts(v, src_len)
        sx_s[0:C, :src_len] = sm
        sx_s[2 * C:3 * C, :src_len] = sp
        sx_s[C:2 * C, :src_len] = v
        acc = jnp.dot(w_ref[:, 0:3 * C], sx_s[:, 0:out_len],
                      preferred_element_type=jnp.float32)
        acc += jnp.dot(w_ref[:, 3 * C:6 * C], sx_s[:, W2:W2 + out_len],
                       preferred_element_type=jnp.float32)
        acc += jnp.dot(w_ref[:, 6 * C:9 * C], sx_s[:, 2 * W2:2 * W2 + out_len],
                       preferred_element_type=jnp.float32)
        return acc + b_ref[...]

    for t in range(nT):
        base = (t * TH + 1) * W2
        ve = xe_ref[:, base:base + Lx]
        vo = xo_ref[:, base:base + Lx]
        qt_s[0 * C:1 * C, :] = _lo16(ve)
        qt_s[1 * C:2 * C, :] = _hi16(ve)
        qt_s[2 * C:3 * C, :] = _lo16(vo)
        qt_s[3 * C:4 * C, :] = _hi16(vo)
        qv = qt_s[...]
        x2v = x2_ref[:, base:base + Lx]

        for branch in range(2):
            if branch == 0:
                w3a, b3a, w3b, b3b = wha_ref, bha_ref, whb_ref, bhb_ref
                x0 = jnp.dot(whq_ref[...], qv,
                             preferred_element_type=jnp.float32)
                x0 = x0 + bh1_ref[...]
            else:
                w3a, b3a, w3b, b3b = wla_ref, bla_ref, wlb_ref, blb_ref
                x0 = jnp.dot(wlq_ref[...], qv,
                             preferred_element_type=jnp.float32)
                x0 += jnp.dot(wx2_ref[...], x2v,
                              preferred_element_type=jnp.float32)
                x0 = x0 + bl1_ref[...]
            x0 = row_mask(x0, t, 1)
            x0_s[...] = x0.astype(bf)

            h1 = _gelu(conv3x3(Lx, L1, w3a, b3a, x0_s))
            h1 = row_mask(h1, t, 2)
            h1_s[...] = h1.astype(bf)

            h2 = _gelu(conv3x3(L1, L2, w3b, b3b, h1_s))
            res = x0_s[:, 2 * W2:2 * W2 + L2].astype(jnp.float32)
            out = h2 + res
            if branch == 0:
                hi_s[...] = out.astype(bf)
            else:
                lo_s[...] = out

        hi = hi_s[...]
        lo2 = 0.5 * lo_s[...]
        obase = t * L2
        y = []
        for k in range(4):
            yk = jnp.dot(whd_ref[k * C:(k + 1) * C, :], hi,
                         preferred_element_type=jnp.float32)
            y.append((yk + bhd_ref[k * C:(k + 1) * C] + lo2).astype(bf))
        o_ref[0, :, obase:obase + L2] = _pack16(y[0], y[1])
        o_ref[1, :, obase:obase + L2] = _pack16(y[2], y[3])


@jax.jit
def _forward(x1, x2, params):
    B, C, H, W = x1.shape
    assert H % 2 == 0 and W % 2 == 0
    H2, W2 = H // 2, W // 2
    assert W2 % 128 == 0 and H2 % 8 == 0
    bf = jnp.bfloat16

    TH = next(d for d in range(min(32, H2), 0, -1) if H2 % d == 0)
    nT = H2 // TH
    LF = (H2 + 6) * W2

    x1i = lax.bitcast_convert_type(
        x1.astype(bf).reshape(B, C, H, W2, 2), jnp.int32)
    x1p = jnp.pad(x1i, ((0, 0), (0, 0), (6, 6), (0, 0)))
    x1t = x1p.reshape(B, C, H2 + 6, 2, W2).transpose(0, 3, 1, 2, 4)
    x1t = x1t.reshape(B, 2, C, LF)

    x2l = x2
    if x2l.shape[2] != H2:
        x2l = jnp.pad(x2l, ((0, 0), (0, 0), (1, 0), (0, 0)))
    x2t = jnp.pad(x2l.astype(bf), ((0, 0), (0, 0), (3, 3), (0, 0)))
    x2t = x2t.reshape(B, C, LF)

    wh = params["convh1_w"]
    Wlh, Whl, Whh = wh[:, :C], wh[:, C:2 * C], wh[:, 2 * C:]
    w_h_quad = 0.5 * jnp.concatenate(
        [Wlh + Whl + Whh, Wlh - Whl - Whh, -Wlh + Whl - Whh, -Wlh - Whl + Whh],
        axis=1)
    wl = params["convl_w"]
    Wll, Wx2 = wl[:, :C], wl[:, C:]
    w_l_quad = jnp.concatenate([0.5 * Wll] * 4, axis=1)

    wh2 = params["convh2_w"]
    G1, G2, G3 = wh2[:C], wh2[C:2 * C], wh2[2 * C:]
    w_head = 0.5 * jnp.concatenate(
        [G1 + G2 + G3, G1 - G2 - G3, -G1 + G2 - G3, -G1 - G2 + G3], axis=0)
    bh = params["convh2_b"]
    g1, g2, g3 = bh[:C], bh[C:2 * C], bh[2 * C:]
    b_head = 0.5 * jnp.concatenate(
        [g1 + g2 + g3, g1 - g2 - g3, -g1 + g2 - g3, -g1 - g2 + g3], axis=0)

    wargs = [w_h_quad.astype(bf), w_l_quad.astype(bf), Wx2.astype(bf),
             params["convh1_b"].reshape(C, 1), params["convl_b"].reshape(C, 1),
             _w3x3_to_mat(params["high_w1"]).astype(bf),
             params["high_b1"].reshape(C, 1),
             _w3x3_to_mat(params["high_w2"]).astype(bf),
             params["high_b2"].reshape(C, 1),
             _w3x3_to_mat(params["low_w1"]).astype(bf),
             params["low_b1"].reshape(C, 1),
             _w3x3_to_mat(params["low_w2"]).astype(bf),
             params["low_b2"].reshape(C, 1),
             w_head.astype(bf), b_head.reshape(4 * C, 1)]

    Lx = (TH + 4) * W2
    L1 = (TH + 2) * W2
    in_specs = [
        pl.BlockSpec((None, None, C, LF), lambda b: (b, 0, 0, 0)),
        pl.BlockSpec((None, None, C, LF), lambda b: (b, 1, 0, 0)),
        pl.BlockSpec((None, C, LF), lambda b: (b, 0, 0)),
    ] + [pl.BlockSpec(w.shape, lambda b: (0, 0)) for w in wargs]

    body = functools.partial(_fused_kernel, C=C, TH=TH, nT=nT, W2=W2, H2=H2)
    yi = pl.pallas_call(
        body,
        out_shape=jax.ShapeDtypeStruct((B, 2, C, H2 * W2), jnp.int32),
        grid=(B,),
        in_specs=in_specs,
        out_specs=pl.BlockSpec((None, 2, C, H2 * W2), lambda b: (b, 0, 0, 0)),
        scratch_shapes=[
            pltpu.VMEM((4 * C, Lx), bf),
            pltpu.VMEM((C, Lx), bf),
            pltpu.VMEM((3 * C, Lx), bf),
            pltpu.VMEM((C, L1), bf),
            pltpu.VMEM((C, TH * W2), bf),
            pltpu.VMEM((C, TH * W2), jnp.float32),
        ],
        compiler_params=pltpu.CompilerParams(
            dimension_semantics=("parallel",),
            vmem_limit_bytes=64 << 20),
    )(x1t, x1t, x2t, *wargs)

    y = yi.reshape(B, 2, C, H2, W2).transpose(0, 2, 3, 1, 4)
    y = lax.bitcast_convert_type(y, bf).reshape(B, C, H, W)
    return y.astype(x1.dtype)


def kernel(x1, x2, convh1_w, convh1_b, high_w1, high_b1, high_w2, high_b2,
           convh2_w, convh2_b, convl_w, convl_b, low_w1, low_b1, low_w2,
           low_b2):
    params = {
        "convh1_w": convh1_w, "convh1_b": convh1_b,
        "high_w1": high_w1, "high_b1": high_b1,
        "high_w2": high_w2, "high_b2": high_b2,
        "convh2_w": convh2_w, "convh2_b": convh2_b,
        "convl_w": convl_w, "convl_b": convl_b,
        "low_w1": low_w1, "low_b1": low_b1,
        "low_w2": low_w2, "low_b2": low_b2,
    }
    return _forward(x1, x2, params)


# speedup vs baseline: 1.2310x; 1.2310x over previous
import functools

import jax
import jax.numpy as jnp
from jax import lax
from jax.experimental import pallas as pl
from jax.experimental.pallas import tpu as pltpu

_GELU_C = (0.3989401215833544, -1.0637133263229617, 2.55036139654138,
           -4.832030293421218, 7.364302003605586, -9.075135234692333,
           8.861428611075373, -6.545424961401749, 3.391661863797997,
           -1.0852160629273473, 0.15981801221487357)


def _gelu(x):
    s = x * x * 0.0625
    p = jnp.float32(_GELU_C[-1])
    for c in _GELU_C[-2::-1]:
        p = p * s + c
    core = 0.5 * x + (x * x) * p
    return jnp.where(x > 4.0, x, jnp.where(x < -4.0, 0.0, core))


def _w3x3_to_mat(w):
    co = w.shape[0]
    return jnp.transpose(w, (0, 2, 3, 1)).reshape(co, 9 * w.shape[1])


def _fused_kernel(qt_ref, x2_ref, whq_ref, wlq_ref, wx2_ref, bh1_ref,
                  bl1_ref, wha_ref, bha_ref, whb_ref, bhb_ref, wla_ref,
                  bla_ref, wlb_ref, blb_ref, whd_ref, bhd_ref, o_ref,
                  x0_s, sx_s, h1_s, hi_s, lo_s, *, C, TH, nT, W2, H2):
    Lx = (TH + 4) * W2
    L1 = (TH + 2) * W2
    L2 = TH * W2
    bf = jnp.bfloat16

    col = lax.broadcasted_iota(jnp.int32, (1, Lx), 1) % W2
    cm_m = (col != 0).astype(bf)
    cm_p = (col != W2 - 1).astype(bf)
    frow = lax.broadcasted_iota(jnp.int32, (1, Lx), 1) // W2

    def row_mask(x, t, start_f):
        if 0 < t < nT - 1:
            return x
        n = x.shape[1] // W2
        orig = t * TH + start_f - 3 + frow[:, :n * W2]
        return jnp.where((orig >= 0) & (orig < H2), x, 0.0)

    def shifts(v, L):
        sm = jnp.concatenate([v[:, :1], v[:, :-1]], axis=-1) * cm_m[:, :L]
        sp = jnp.concatenate([v[:, 1:], v[:, -1:]], axis=-1) * cm_p[:, :L]
        return sm, sp

    def conv3x3(src_len, out_len, w_ref, b_ref, src_sref):
        v = src_sref[:, : src_len]
        sm, sp = shifts(v, src_len)
        sx_s[0:C, :src_len] = sm
        sx_s[2 * C:3 * C, :src_len] = sp
        sx_s[C:2 * C, :src_len] = v
        acc = jnp.dot(w_ref[:, 0:3 * C], sx_s[:, 0:out_len],
                      preferred_element_type=jnp.float32)
        acc += jnp.dot(w_ref[:, 3 * C:6 * C], sx_s[:, W2:W2 + out_len],
                       preferred_element_type=jnp.float32)
        acc += jnp.dot(w_ref[:, 6 * C:9 * C], sx_s[:, 2 * W2:2 * W2 + out_len],
                       preferred_element_type=jnp.float32)
        return acc + b_ref[...]

    for t in range(nT):
        base = (t * TH + 1) * W2
        qv = qt_ref[:, base:base + Lx]
        x2v = x2_ref[:, base:base + Lx]

        for branch in range(2):
            if branch == 0:
                w3a, b3a, w3b, b3b = wha_ref, bha_ref, whb_ref, bhb_ref
                x0 = jnp.dot(whq_ref[...], qv,
                             preferred_element_type=jnp.float32)
                x0 = x0 + bh1_ref[...]
            else:
                w3a, b3a, w3b, b3b = wla_ref, bla_ref, wlb_ref, blb_ref
                x0 = jnp.dot(wlq_ref[...], qv,
                             preferred_element_type=jnp.float32)
                x0 += jnp.dot(wx2_ref[...], x2v,
                              preferred_element_type=jnp.float32)
                x0 = x0 + bl1_ref[...]
            x0 = row_mask(x0, t, 1)
            x0_s[...] = x0.astype(bf)

            h1 = _gelu(conv3x3(Lx, L1, w3a, b3a, x0_s))
            h1 = row_mask(h1, t, 2)
            h1_s[...] = h1.astype(bf)

            h2 = _gelu(conv3x3(L1, L2, w3b, b3b, h1_s))
            res = x0_s[:, 2 * W2:2 * W2 + L2].astype(jnp.float32)
            out = h2 + res
            if branch == 0:
                hi_s[...] = out.astype(bf)
            else:
                lo_s[...] = out

        hi = hi_s[...]
        lo2 = 0.5 * lo_s[...]
        obase = t * L2
        for k in range(4):
            yk = jnp.dot(whd_ref[k * C:(k + 1) * C, :], hi,
                         preferred_element_type=jnp.float32)
            yk = yk + bhd_ref[k * C:(k + 1) * C] + lo2
            o_ref[k * C:(k + 1) * C, obase:obase + L2] = yk.astype(o_ref.dtype)


@jax.jit
def _forward(x1, x2, params):
    B, C, H, W = x1.shape
    assert H % 2 == 0 and W % 2 == 0
    H2, W2 = H // 2, W // 2
    assert W2 % 128 == 0 and H2 % 8 == 0
    bf = jnp.bfloat16

    TH = next(d for d in range(min(32, H2), 0, -1) if H2 % d == 0)
    nT = H2 // TH
    LF = (H2 + 6) * W2

    x1p = jnp.pad(x1.astype(bf), ((0, 0), (0, 0), (6, 6), (0, 0)))
    q = x1p.reshape(B, C, H2 + 6, 2, W2, 2).transpose(0, 3, 5, 1, 2, 4)
    qt = q.reshape(B, 4 * C, LF)

    x2l = x2
    if x2l.shape[2] != H2:
        x2l = jnp.pad(x2l, ((0, 0), (0, 0), (1, 0), (0, 0)))
    x2t = jnp.pad(x2l.astype(bf), ((0, 0), (0, 0), (3, 3), (0, 0)))
    x2t = x2t.reshape(B, C, LF)

    wh = params["convh1_w"]
    Wlh, Whl, Whh = wh[:, :C], wh[:, C:2 * C], wh[:, 2 * C:]
    w_h_quad = 0.5 * jnp.concatenate(
        [Wlh + Whl + Whh, Wlh - Whl - Whh, -Wlh + Whl - Whh, -Wlh - Whl + Whh],
        axis=1)
    wl = params["convl_w"]
    Wll, Wx2 = wl[:, :C], wl[:, C:]
    w_l_quad = jnp.concatenate([0.5 * Wll] * 4, axis=1)

    wh2 = params["convh2_w"]
    G1, G2, G3 = wh2[:C], wh2[C:2 * C], wh2[2 * C:]
    w_head = 0.5 * jnp.concatenate(
        [G1 + G2 + G3, G1 - G2 - G3, -G1 + G2 - G3, -G1 - G2 + G3], axis=0)
    bh = params["convh2_b"]
    g1, g2, g3 = bh[:C], bh[C:2 * C], bh[2 * C:]
    b_head = 0.5 * jnp.concatenate(
        [g1 + g2 + g3, g1 - g2 - g3, -g1 + g2 - g3, -g1 - g2 + g3], axis=0)

    wargs = [w_h_quad.astype(bf), w_l_quad.astype(bf), Wx2.astype(bf),
             params["convh1_b"].reshape(C, 1), params["convl_b"].reshape(C, 1),
             _w3x3_to_mat(params["high_w1"]).astype(bf),
             params["high_b1"].reshape(C, 1),
             _w3x3_to_mat(params["high_w2"]).astype(bf),
             params["high_b2"].reshape(C, 1),
             _w3x3_to_mat(params["low_w1"]).astype(bf),
             params["low_b1"].reshape(C, 1),
             _w3x3_to_mat(params["low_w2"]).astype(bf),
             params["low_b2"].reshape(C, 1),
             w_head.astype(bf), b_head.reshape(4 * C, 1)]

    Lx = (TH + 4) * W2
    L1 = (TH + 2) * W2
    in_specs = [
        pl.BlockSpec((None, 4 * C, LF), lambda b: (b, 0, 0)),
        pl.BlockSpec((None, C, LF), lambda b: (b, 0, 0)),
    ] + [pl.BlockSpec(w.shape, lambda b: (0, 0)) for w in wargs]

    body = functools.partial(_fused_kernel, C=C, TH=TH, nT=nT, W2=W2, H2=H2)
    quads = pl.pallas_call(
        body,
        out_shape=jax.ShapeDtypeStruct((B, 4 * C, H2 * W2), bf),
        grid=(B,),
        in_specs=in_specs,
        out_specs=pl.BlockSpec((None, 4 * C, H2 * W2), lambda b: (b, 0, 0)),
        scratch_shapes=[
            pltpu.VMEM((C, Lx), bf),
            pltpu.VMEM((3 * C, Lx), bf),
            pltpu.VMEM((C, L1), bf),
            pltpu.VMEM((C, TH * W2), bf),
            pltpu.VMEM((C, TH * W2), jnp.float32),
        ],
        compiler_params=pltpu.CompilerParams(
            dimension_semantics=("parallel",),
            vmem_limit_bytes=64 << 20),
    )(qt, x2t, *wargs)

    y = quads.reshape(B, 2, 2, C, H2, W2).transpose(0, 3, 4, 1, 5, 2)
    return y.reshape(B, C, H, W).astype(x1.dtype)


def kernel(x1, x2, convh1_w, convh1_b, high_w1, high_b1, high_w2, high_b2,
           convh2_w, convh2_b, convl_w, convl_b, low_w1, low_b1, low_w2,
           low_b2):
    params = {
        "convh1_w": convh1_w, "convh1_b": convh1_b,
        "high_w1": high_w1, "high_b1": high_b1,
        "high_w2": high_w2, "high_b2": high_b2,
        "convh2_w": convh2_w, "convh2_b": convh2_b,
        "convl_w": convl_w, "convl_b": convl_b,
        "low_w1": low_w1, "low_b1": low_b1,
        "low_w2": low_w2, "low_b2": low_b2,
    }
    return _forward(x1, x2, params)


# speedup vs baseline: 1.2361x; 1.0042x over previous
import functools

import jax
import jax.numpy as jnp
from jax import lax
from jax.experimental import pallas as pl
from jax.experimental.pallas import tpu as pltpu

_SQRT1_2 = 0.7071067811865476


def _erf(x):
    a1, a2, a3, a4, a5 = (0.254829592, -0.284496736, 1.421413741,
                          -1.453152027, 1.061405429)
    p = 0.3275911
    ax = jnp.abs(x)
    t = 1.0 / (1.0 + p * ax)
    poly = ((((a5 * t + a4) * t + a3) * t + a2) * t + a1) * t
    y = 1.0 - poly * jnp.exp(-ax * ax)
    return jnp.sign(x) * y


def _gelu(x):
    return 0.5 * x * (1.0 + _erf(x * _SQRT1_2))


def _w3x3_to_mat(w):
    co = w.shape[0]
    return jnp.transpose(w, (0, 2, 3, 1)).reshape(co, 9 * w.shape[1])


def _fused_kernel(qt_ref, x2_ref, whq_ref, wlq_ref, wx2_ref, bh1_ref,
                  bl1_ref, wha_ref, bha_ref, whb_ref, bhb_ref, wla_ref,
                  bla_ref, wlb_ref, blb_ref, whd_ref, bhd_ref, o_ref,
                  x0_s, sx_s, h1_s, hi_s, lo_s, *, C, TH, nT, W2, H2):
    Lx = (TH + 4) * W2
    L1 = (TH + 2) * W2
    L2 = TH * W2
    bf = jnp.bfloat16

    col = lax.broadcasted_iota(jnp.int32, (1, Lx), 1) % W2
    cm_m = (col != 0).astype(bf)
    cm_p = (col != W2 - 1).astype(bf)
    frow = lax.broadcasted_iota(jnp.int32, (1, Lx), 1) // W2

    def row_mask(x, t, start_f):
        if 0 < t < nT - 1:
            return x
        n = x.shape[1] // W2
        orig = t * TH + start_f - 3 + frow[:, :n * W2]
        return jnp.where((orig >= 0) & (orig < H2), x, 0.0)

    def shifts(v, L):
        sm = jnp.concatenate([v[:, :1], v[:, :-1]], axis=-1) * cm_m[:, :L]
        sp = jnp.concatenate([v[:, 1:], v[:, -1:]], axis=-1) * cm_p[:, :L]
        return sm, sp

    def conv3x3(src_len, out_len, w_ref, b_ref, src_sref):
        v = src_sref[:, : src_len]
        sm, sp = shifts(v, src_len)
        sx_s[0:C, :src_len] = sm
        sx_s[2 * C:3 * C, :src_len] = sp
        sx_s[C:2 * C, :src_len] = v
        acc = jnp.dot(w_ref[:, 0:3 * C], sx_s[:, 0:out_len],
                      preferred_element_type=jnp.float32)
        acc += jnp.dot(w_ref[:, 3 * C:6 * C], sx_s[:, W2:W2 + out_len],
                       preferred_element_type=jnp.float32)
        acc += jnp.dot(w_ref[:, 6 * C:9 * C], sx_s[:, 2 * W2:2 * W2 + out_len],
                       preferred_element_type=jnp.float32)
        return acc + b_ref[...]

    for t in range(nT):
        base = (t * TH + 1) * W2
        qv = qt_ref[:, base:base + Lx]
        x2v = x2_ref[:, base:base + Lx]

        for branch in range(2):
            if branch == 0:
                w3a, b3a, w3b, b3b = wha_ref, bha_ref, whb_ref, bhb_ref
                x0 = jnp.dot(whq_ref[...], qv,
                             preferred_element_type=jnp.float32)
                x0 = x0 + bh1_ref[...]
            else:
                w3a, b3a, w3b, b3b = wla_ref, bla_ref, wlb_ref, blb_ref
                x0 = jnp.dot(wlq_ref[...], qv,
                             preferred_element_type=jnp.float32)
                x0 += jnp.dot(wx2_ref[...], x2v,
                              preferred_element_type=jnp.float32)
                x0 = x0 + bl1_ref[...]
            x0 = row_mask(x0, t, 1)
            x0_s[...] = x0.astype(bf)

            h1 = _gelu(conv3x3(Lx, L1, w3a, b3a, x0_s))
            h1 = row_mask(h1, t, 2)
            h1_s[...] = h1.astype(bf)

            h2 = _gelu(conv3x3(L1, L2, w3b, b3b, h1_s))
            res = x0_s[:, 2 * W2:2 * W2 + L2].astype(jnp.float32)
            out = h2 + res
            if branch == 0:
                hi_s[...] = out.astype(bf)
            else:
                lo_s[...] = out

        hi = hi_s[...]
        lo2 = 0.5 * lo_s[...]
        obase = t * L2
        for k in range(4):
            yk = jnp.dot(whd_ref[k * C:(k + 1) * C, :], hi,
                         preferred_element_type=jnp.float32)
            yk = yk + bhd_ref[k * C:(k + 1) * C] + lo2
            o_ref[k * C:(k + 1) * C, obase:obase + L2] = yk.astype(o_ref.dtype)


@jax.jit
def _forward(x1, x2, params):
    B, C, H, W = x1.shape
    assert H % 2 == 0 and W % 2 == 0
    H2, W2 = H // 2, W // 2
    assert W2 % 128 == 0 and H2 % 8 == 0
    bf = jnp.bfloat16

    TH = next(d for d in range(min(32, H2), 0, -1) if H2 % d == 0)
    nT = H2 // TH
    LF = (H2 + 6) * W2

    x1p = jnp.pad(x1.astype(bf), ((0, 0), (0, 0), (6, 6), (0, 0)))
    q = x1p.reshape(B, C, H2 + 6, 2, W2, 2).transpose(0, 3, 5, 1, 2, 4)
    qt = q.reshape(B, 4 * C, LF)

    x2l = x2
    if x2l.shape[2] != H2:
        x2l = jnp.pad(x2l, ((0, 0), (0, 0), (1, 0), (0, 0)))
    x2t = jnp.pad(x2l.astype(bf), ((0, 0), (0, 0), (3, 3), (0, 0)))
    x2t = x2t.reshape(B, C, LF)

    wh = params["convh1_w"]
    Wlh, Whl, Whh = wh[:, :C], wh[:, C:2 * C], wh[:, 2 * C:]
    w_h_quad = 0.5 * jnp.concatenate(
        [Wlh + Whl + Whh, Wlh - Whl - Whh, -Wlh + Whl - Whh, -Wlh - Whl + Whh],
        axis=1)
    wl = params["convl_w"]
    Wll, Wx2 = wl[:, :C], wl[:, C:]
    w_l_quad = jnp.concatenate([0.5 * Wll] * 4, axis=1)

    wh2 = params["convh2_w"]
    G1, G2, G3 = wh2[:C], wh2[C:2 * C], wh2[2 * C:]
    w_head = 0.5 * jnp.concatenate(
        [G1 + G2 + G3, G1 - G2 - G3, -G1 + G2 - G3, -G1 - G2 + G3], axis=0)
    bh = params["convh2_b"]
    g1, g2, g3 = bh[:C], bh[C:2 * C], bh[2 * C:]
    b_head = 0.5 * jnp.concatenate(
        [g1 + g2 + g3, g1 - g2 - g3, -g1 + g2 - g3, -g1 - g2 + g3], axis=0)

    wargs = [w_h_quad.astype(bf), w_l_quad.astype(bf), Wx2.astype(bf),
             params["convh1_b"].reshape(C, 1), params["convl_b"].reshape(C, 1),
             _w3x3_to_mat(params["high_w1"]).astype(bf),
             params["high_b1"].reshape(C, 1),
             _w3x3_to_mat(params["high_w2"]).astype(bf),
             params["high_b2"].reshape(C, 1),
             _w3x3_to_mat(params["low_w1"]).astype(bf),
             params["low_b1"].reshape(C, 1),
             _w3x3_to_mat(params["low_w2"]).astype(bf),
             params["low_b2"].reshape(C, 1),
             w_head.astype(bf), b_head.reshape(4 * C, 1)]

    Lx = (TH + 4) * W2
    L1 = (TH + 2) * W2
    in_specs = [
        pl.BlockSpec((None, 4 * C, LF), lambda b: (b, 0, 0)),
        pl.BlockSpec((None, C, LF), lambda b: (b, 0, 0)),
    ] + [pl.BlockSpec(w.shape, lambda b: (0, 0)) for w in wargs]

    body = functools.partial(_fused_kernel, C=C, TH=TH, nT=nT, W2=W2, H2=H2)
    quads = pl.pallas_call(
        body,
        out_shape=jax.ShapeDtypeStruct((B, 4 * C, H2 * W2), bf),
        grid=(B,),
        in_specs=in_specs,
        out_specs=pl.BlockSpec((None, 4 * C, H2 * W2), lambda b: (b, 0, 0)),
        scratch_shapes=[
            pltpu.VMEM((C, Lx), bf),
            pltpu.VMEM((3 * C, Lx), bf),
            pltpu.VMEM((C, L1), bf),
            pltpu.VMEM((C, TH * W2), bf),
            pltpu.VMEM((C, TH * W2), jnp.float32),
        ],
        compiler_params=pltpu.CompilerParams(
            dimension_semantics=("parallel",),
            vmem_limit_bytes=64 << 20),
    )(qt, x2t, *wargs)

    y = quads.reshape(B, 2, 2, C, H2, W2).transpose(0, 3, 4, 1, 5, 2)
    return y.reshape(B, C, H, W).astype(x1.dtype)


def kernel(x1, x2, convh1_w, convh1_b, high_w1, high_b1, high_w2, high_b2,
           convh2_w, convh2_b, convl_w, convl_b, low_w1, low_b1, low_w2,
           low_b2):
    params = {
        "convh1_w": convh1_w, "convh1_b": convh1_b,
        "high_w1": high_w1, "high_b1": high_b1,
        "high_w2": high_w2, "high_b2": high_b2,
        "convh2_w": convh2_w, "convh2_b": convh2_b,
        "convl_w": convl_w, "convl_b": convl_b,
        "low_w1": low_w1, "low_b1": low_b1,
        "low_w2": low_w2, "low_b2": low_b2,
    }
    return _forward(x1, x2, params)
